# Initial kernel scaffold; baseline (speedup 1.0000x reference)
import jax
import jax.numpy as jnp
from jax.experimental import pallas as pl


def kernel(visual_tokens, text_tokens, v_ln_g, v_ln_b, v_W, v_b, t_ln_g, t_ln_b, t_W, t_b, o_W1, o_b1, o_W2, o_b2, o_W3, o_b3):
    raise NotImplementedError("write your pallas kernel here")


# speedup vs baseline: 3.5156x; 3.5156x over previous
import jax
import jax.numpy as jnp
from jax.experimental import pallas as pl
from jax.experimental.pallas import tpu as pltpu

D = 2048
H = 512
B = 4
N = 2048
BN = B * N
TOPK = BN // 4
CH = 512
PROWS = BN // CH
EPS = 1e-5


def _ln(x, g, b):
    mu = jnp.mean(x, axis=-1, keepdims=True)
    var = jnp.mean((x - mu) ** 2, axis=-1, keepdims=True)
    return (x - mu) / jnp.sqrt(var + EPS) * g + b


def _gelu(x):
    return 0.5 * x * (1.0 + jax.lax.erf(x * (2.0 ** -0.5)))


def _text_kernel(text_ref, g_ref, b_ref, tW_ref, tb_ref, W1t_ref, b1_ref,
                 ct_ref):
    xn = _ln(text_ref[...], g_ref[...], b_ref[...])
    t = _gelu(jnp.dot(xn, tW_ref[...], preferred_element_type=jnp.float32)
              + tb_ref[...])
    tm = jnp.mean(t, axis=0, keepdims=True)
    ct_ref[...] = (jnp.dot(tm, W1t_ref[...],
                           preferred_element_type=jnp.float32)
                   + b1_ref[...][None, :])


def _scorer_kernel(vis_ref, g_ref, b_ref, vW_ref, vb_ref, W1v_ref, W1f_ref,
                   ct_ref, W2_ref, b2_ref, W3_ref, b3_ref, probs_ref, v_scr):
    bidx = pl.program_id(0)
    acc = jnp.zeros((1, H), jnp.float32)
    for c in range(N // CH):
        x = vis_ref[0, pl.ds(c * CH, CH), :]
        xn = _ln(x, g_ref[...], b_ref[...])
        vc = _gelu(jnp.dot(xn, vW_ref[...],
                           preferred_element_type=jnp.float32) + vb_ref[...])
        v_scr[pl.ds(c * CH, CH), :] = vc
        acc = acc + jnp.sum(vc, axis=0, keepdims=True)
    vf = acc / jnp.float32(N)
    cb = (jnp.dot(vf, W1f_ref[...], preferred_element_type=jnp.float32)
          + ct_ref[...])
    for c in range(N // CH):
        vc = v_scr[pl.ds(c * CH, CH), :]
        h1 = _gelu(jnp.dot(vc, W1v_ref[...],
                           preferred_element_type=jnp.float32) + cb)
        h2 = _gelu(jnp.dot(h1, W2_ref[...],
                           preferred_element_type=jnp.float32) + b2_ref[...])
        zT = jax.lax.dot_general(
            W3_ref[...], h2, (((0,), (1,)), ((), ())),
            preferred_element_type=jnp.float32) + b3_ref[...][:, None]
        m = jnp.maximum(zT[0:1], zT[1:2])
        s0 = zT[0:1] - m
        s1 = zT[1:2] - m
        lse = jnp.log(jnp.exp(s0) + jnp.exp(s1))
        sc0 = s0 - lse
        sc1 = s1 - lse
        m2 = jnp.maximum(sc0, sc1)
        e0 = jnp.exp(sc0 - m2)
        e1 = jnp.exp(sc1 - m2)
        p = e0 / (e0 + e1)
        probs_ref[pl.ds(bidx * (N // CH) + c, 1), :] = p


def _topk_kernel(probs_ref, mask_ref):
    bits = jax.lax.bitcast_convert_type(probs_ref[...], jnp.int32)
    kk = jnp.int32(TOPK)

    def cnt_gt(t):
        return jnp.sum((bits > t).astype(jnp.int32))

    def val_body(_, lohi):
        lo, hi = lohi
        mid = (lo + hi) >> 1
        pred = cnt_gt(mid) >= kk
        return jnp.where(pred, mid, lo), jnp.where(pred, hi, mid)

    lo, hi = jax.lax.fori_loop(
        0, 31, val_body, (jnp.int32(-1), jnp.int32(0x3F800000)))
    thr = hi
    m = kk - cnt_gt(thr)
    rows = jax.lax.broadcasted_iota(jnp.int32, (PROWS, CH), 0)
    cols = jax.lax.broadcasted_iota(jnp.int32, (PROWS, CH), 1)
    idx = rows * CH + cols
    eq = bits == thr

    def idx_body(_, lohi):
        lo, hi = lohi
        mid = (lo + hi) >> 1
        cm = jnp.sum((eq & (idx < mid)).astype(jnp.int32))
        pred = cm >= m
        return jnp.where(pred, lo, mid), jnp.where(pred, mid, hi)

    lo2, hi2 = jax.lax.fori_loop(
        0, 14, idx_body, (jnp.int32(-1), jnp.int32(BN)))
    mask_ref[...] = (bits > thr) | (eq & (idx < hi2))


def kernel(visual_tokens, text_tokens, v_ln_g, v_ln_b, v_W, v_b, t_ln_g,
           t_ln_b, t_W, t_b, o_W1, o_b1, o_W2, o_b2, o_W3, o_b3):
    W1v = o_W1[0:H]
    W1f = o_W1[H:2 * H]
    W1t = o_W1[2 * H:3 * H]

    ct = pl.pallas_call(
        _text_kernel,
        out_shape=jax.ShapeDtypeStruct((1, H), jnp.float32),
    )(text_tokens, t_ln_g, t_ln_b, t_W, t_b, W1t, o_b1)

    full = lambda a: pl.BlockSpec(a.shape, lambda b: (0,) * a.ndim)
    probs = pl.pallas_call(
        _scorer_kernel,
        grid=(B,),
        in_specs=[
            pl.BlockSpec((1, N, D), lambda b: (b, 0, 0)),
            full(v_ln_g), full(v_ln_b), full(v_W), full(v_b),
            full(W1v), full(W1f), full(ct),
            full(o_W2), full(o_b2), full(o_W3), full(o_b3),
        ],
        out_specs=pl.BlockSpec((PROWS, CH), lambda b: (0, 0)),
        out_shape=jax.ShapeDtypeStruct((PROWS, CH), jnp.float32),
        scratch_shapes=[pltpu.VMEM((N, H), jnp.float32)],
    )(visual_tokens, v_ln_g, v_ln_b, v_W, v_b, W1v, W1f, ct,
      o_W2, o_b2, o_W3, o_b3)

    mask = pl.pallas_call(
        _topk_kernel,
        out_shape=jax.ShapeDtypeStruct((PROWS, CH), jnp.bool_),
    )(probs)
    return mask.reshape(BN)
